# baseline (speedup 1.0000x reference)
import jax
import jax.numpy as jnp
from jax.experimental import pallas as pl
from jax.experimental.pallas import tpu as pltpu

N = 10000
D_IN = 128
D_OUT = 128
BM = 400


def _fts_kernel(x_ref, w_ref, fts_ref):
    fts_ref[...] = jax.lax.dot_general(
        x_ref[...], w_ref[...],
        dimension_numbers=(((1,), (1,)), ((), ())),
        preferred_element_type=jnp.float32,
    ).astype(jnp.bfloat16)


def _agg_kernel(fts_ref, b_ref, a_ref, adj_ref, out_ref):
    r = jnp.dot(
        adj_ref[...].astype(jnp.bfloat16), fts_ref[...],
        preferred_element_type=jnp.float32,
    ) + b_ref[...]
    out_ref[...] = jnp.where(r >= 0, r, a_ref[0, 0] * r)


@jax.jit
def kernel(x, adj_mat, W, bias, prelu_a):
    x2 = jnp.squeeze(x, 0)
    b2 = bias.reshape(1, D_OUT)
    a2 = prelu_a.reshape(1, 1)

    fts = pl.pallas_call(
        _fts_kernel,
        out_shape=jax.ShapeDtypeStruct((N, D_OUT), jnp.bfloat16),
    )(x2, W)

    out = pl.pallas_call(
        _agg_kernel,
        grid=(N // BM,),
        in_specs=[
            pl.BlockSpec((N, D_OUT), lambda m: (0, 0)),
            pl.BlockSpec((1, D_OUT), lambda m: (0, 0)),
            pl.BlockSpec((1, 1), lambda m: (0, 0)),
            pl.BlockSpec((BM, N), lambda m: (m, 0)),
        ],
        out_specs=pl.BlockSpec((BM, D_OUT), lambda m: (m, 0)),
        out_shape=jax.ShapeDtypeStruct((N, D_OUT), jnp.float32),
        compiler_params=pltpu.CompilerParams(
            dimension_semantics=("parallel",),
        ),
    )(fts, b2, a2, adj_mat)

    return out[None, :, :]


# speedup vs baseline: 1.0204x; 1.0204x over previous
import jax
import jax.numpy as jnp
from jax.experimental import pallas as pl
from jax.experimental.pallas import tpu as pltpu

N = 10000
D_IN = 128
D_OUT = 128
BM = 200


def _gcn_kernel(x_ref, w_ref, b_ref, a_ref, adj_ref, out_ref, fts_ref):
    m = pl.program_id(0)

    @pl.when(m == 0)
    def _compute_fts():
        fts_ref[...] = jax.lax.dot_general(
            x_ref[...], w_ref[...],
            dimension_numbers=(((1,), (1,)), ((), ())),
            preferred_element_type=jnp.float32,
        ).astype(jnp.bfloat16)

    r = jnp.dot(
        adj_ref[...].astype(jnp.bfloat16), fts_ref[...],
        preferred_element_type=jnp.float32,
    ) + b_ref[...]
    out_ref[...] = jnp.where(r >= 0, r, a_ref[0, 0] * r)


@jax.jit
def kernel(x, adj_mat, W, bias, prelu_a):
    x2 = jnp.squeeze(x, 0)
    b2 = bias.reshape(1, D_OUT)
    a2 = prelu_a.reshape(1, 1)

    out = pl.pallas_call(
        _gcn_kernel,
        grid=(N // BM,),
        in_specs=[
            pl.BlockSpec((N, D_IN), lambda m: (0, 0)),
            pl.BlockSpec((D_OUT, D_IN), lambda m: (0, 0)),
            pl.BlockSpec((1, D_OUT), lambda m: (0, 0)),
            pl.BlockSpec((1, 1), lambda m: (0, 0)),
            pl.BlockSpec((BM, N), lambda m: (m, 0)),
        ],
        out_specs=pl.BlockSpec((BM, D_OUT), lambda m: (m, 0)),
        out_shape=jax.ShapeDtypeStruct((N, D_OUT), jnp.float32),
        scratch_shapes=[pltpu.VMEM((N, D_OUT), jnp.bfloat16)],
        compiler_params=pltpu.CompilerParams(
            dimension_semantics=("arbitrary",),
        ),
    )(x2, W, b2, a2, adj_mat)

    return out[None, :, :]


# speedup vs baseline: 1.0343x; 1.0136x over previous
import jax
import jax.numpy as jnp
from jax.experimental import pallas as pl
from jax.experimental.pallas import tpu as pltpu

N = 10000
D_IN = 128
D_OUT = 128
BM = 400


def _gcn_kernel(x_ref, w_ref, b_ref, a_ref, adj_ref, out_ref, fts_ref):
    m = pl.program_id(0)

    @pl.when(m == 0)
    def _compute_fts():
        fts_ref[...] = jax.lax.dot_general(
            x_ref[...], w_ref[...],
            dimension_numbers=(((1,), (1,)), ((), ())),
            preferred_element_type=jnp.float32,
        ).astype(jnp.bfloat16)

    r = jnp.dot(
        adj_ref[...].astype(jnp.bfloat16), fts_ref[...],
        preferred_element_type=jnp.float32,
    ) + b_ref[...]
    out_ref[...] = jnp.where(r >= 0, r, a_ref[0, 0] * r)


@jax.jit
def kernel(x, adj_mat, W, bias, prelu_a):
    x2 = jnp.squeeze(x, 0)
    b2 = bias.reshape(1, D_OUT)
    a2 = prelu_a.reshape(1, 1)

    out = pl.pallas_call(
        _gcn_kernel,
        grid=(N // BM,),
        in_specs=[
            pl.BlockSpec((N, D_IN), lambda m: (0, 0)),
            pl.BlockSpec((D_OUT, D_IN), lambda m: (0, 0)),
            pl.BlockSpec((1, D_OUT), lambda m: (0, 0)),
            pl.BlockSpec((1, 1), lambda m: (0, 0)),
            pl.BlockSpec((BM, N), lambda m: (m, 0)),
        ],
        out_specs=pl.BlockSpec((BM, D_OUT), lambda m: (m, 0)),
        out_shape=jax.ShapeDtypeStruct((N, D_OUT), jnp.float32),
        scratch_shapes=[pltpu.VMEM((N, D_OUT), jnp.bfloat16)],
        compiler_params=pltpu.CompilerParams(
            dimension_semantics=("arbitrary",),
        ),
    )(x2, W, b2, a2, adj_mat)

    return out[None, :, :]
